# baseline (device time: 65998 ns/iter reference)
import jax
import jax.numpy as jnp
from jax import lax
from jax.experimental import pallas as pl
from jax.experimental.pallas import tpu as pltpu

N_DEV = 4
EPS = 1e-5
BLOCK_M = 512
LANES = 128


def _pack_rows(s, nrows):
    r_idx = lax.broadcasted_iota(jnp.int32, (nrows, LANES), 0)
    b_idx = lax.broadcasted_iota(jnp.int32, (nrows, LANES), 1)
    masked = s * (r_idx % LANES == b_idx).astype(jnp.float32)
    a_idx = lax.broadcasted_iota(jnp.int32, (nrows // LANES, nrows), 0)
    rr_idx = lax.broadcasted_iota(jnp.int32, (nrows // LANES, nrows), 1)
    sel = (rr_idx // LANES == a_idx).astype(jnp.float32)
    return jax.lax.dot(sel, masked, preferred_element_type=jnp.float32)


def _unpack_rows(p, nrows):
    r_idx = lax.broadcasted_iota(jnp.int32, (nrows, LANES), 0)
    b_idx = lax.broadcasted_iota(jnp.int32, (nrows, LANES), 1)
    a_idx = lax.broadcasted_iota(jnp.int32, (nrows, nrows // LANES), 1)
    rr_idx = lax.broadcasted_iota(jnp.int32, (nrows, nrows // LANES), 0)
    sel = (rr_idx // LANES == a_idx).astype(jnp.float32)
    w = jax.lax.dot(sel, p, preferred_element_type=jnp.float32)
    w = w * (r_idx % LANES == b_idx).astype(jnp.float32)
    return jnp.sum(w, axis=1, keepdims=True)


def _fused_body(x_ref, g_ref, out_ref,
                xb_ref, acc_ref, comm_ref, send_sems, recv_sems):
    i = pl.program_id(0)
    nblk = pl.num_programs(0) // 2
    me = lax.axis_index("i")
    pk = BLOCK_M // LANES

    @pl.when(i < nblk)
    def _():
        xf = x_ref[...]
        s = jnp.sum(xf * xf, axis=1, keepdims=True)
        acc_ref[pl.ds(i * pk, pk), :] = _pack_rows(s, BLOCK_M)
        xb_ref[pl.ds(i * BLOCK_M, BLOCK_M), :] = xf.astype(jnp.bfloat16)

    @pl.when(i == nblk - 1)
    def _():
        barrier = pltpu.get_barrier_semaphore()
        for d in range(1, N_DEV):
            peer = (me + d) % N_DEV
            pl.semaphore_signal(
                barrier, inc=1,
                device_id=(peer,), device_id_type=pl.DeviceIdType.MESH,
            )
        pl.semaphore_wait(barrier, N_DEV - 1)

        sends = []
        for d in range(1, N_DEV):
            peer = (me + d) % N_DEV
            rdma = pltpu.make_async_remote_copy(
                src_ref=acc_ref,
                dst_ref=comm_ref.at[me],
                send_sem=send_sems.at[d - 1],
                recv_sem=recv_sems.at[me],
                device_id=(peer,),
                device_id_type=pl.DeviceIdType.MESH,
            )
            rdma.start()
            sends.append(rdma)

        for d in range(1, N_DEV):
            src = (me - d + N_DEV) % N_DEV
            recv = pltpu.make_async_remote_copy(
                src_ref=acc_ref,
                dst_ref=comm_ref.at[src],
                send_sem=send_sems.at[0],
                recv_sem=recv_sems.at[src],
                device_id=(me,),
                device_id_type=pl.DeviceIdType.MESH,
            )
            recv.wait_recv()

        total = acc_ref[...]
        for peer in range(N_DEV):
            total = total + jnp.where(me == peer, 0.0, comm_ref[peer, :, :])

        for rdma in sends:
            rdma.wait_send()

        acc_ref[...] = lax.rsqrt(total * (1.0 / (N_DEV * 2048.0)) + EPS)

    @pl.when(i >= nblk)
    def _():
        j = i - nblk
        rp = acc_ref[pl.ds(j * pk, pk), :]
        u = _unpack_rows(rp, BLOCK_M).astype(jnp.bfloat16)
        xb = xb_ref[pl.ds(j * BLOCK_M, BLOCK_M), :]
        out_ref[...] = xb * u * g_ref[...].astype(jnp.bfloat16)


def kernel(x, gamma):
    m, n_loc = x.shape
    nblk = m // BLOCK_M

    g2 = gamma.reshape(1, n_loc)

    out = pl.pallas_call(
        _fused_body,
        grid=(2 * nblk,),
        out_shape=jax.ShapeDtypeStruct((m, n_loc), jnp.bfloat16),
        in_specs=[
            pl.BlockSpec((BLOCK_M, n_loc),
                         lambda i: (jnp.minimum(i, nblk - 1), 0),
                         memory_space=pltpu.VMEM),
            pl.BlockSpec((1, n_loc), lambda i: (0, 0),
                         memory_space=pltpu.VMEM),
        ],
        out_specs=pl.BlockSpec((BLOCK_M, n_loc),
                               lambda i: (jnp.where(i < nblk, 0, i - nblk), 0),
                               memory_space=pltpu.VMEM),
        scratch_shapes=[
            pltpu.VMEM((m, n_loc), jnp.bfloat16),
            pltpu.VMEM((m // LANES, LANES), jnp.float32),
            pltpu.VMEM((N_DEV, m // LANES, LANES), jnp.float32),
            pltpu.SemaphoreType.DMA((N_DEV - 1,)),
            pltpu.SemaphoreType.DMA((N_DEV,)),
        ],
        compiler_params=pltpu.CompilerParams(
            collective_id=0, vmem_limit_bytes=56 * 1024 * 1024
        ),
    )(x, g2)
    return out


# device time: 61250 ns/iter; 1.0775x vs baseline; 1.0775x over previous
import jax
import jax.numpy as jnp
from jax import lax
from jax.experimental import pallas as pl
from jax.experimental.pallas import tpu as pltpu

N_DEV = 4
EPS = 1e-5
BLOCK_M = 1024
LANES = 128


def _pack_rows(s, nrows):
    r_idx = lax.broadcasted_iota(jnp.int32, (nrows, LANES), 0)
    b_idx = lax.broadcasted_iota(jnp.int32, (nrows, LANES), 1)
    masked = s * (r_idx % LANES == b_idx).astype(jnp.float32)
    a_idx = lax.broadcasted_iota(jnp.int32, (nrows // LANES, nrows), 0)
    rr_idx = lax.broadcasted_iota(jnp.int32, (nrows // LANES, nrows), 1)
    sel = (rr_idx // LANES == a_idx).astype(jnp.float32)
    return jax.lax.dot(sel, masked, preferred_element_type=jnp.float32)


def _unpack_rows(p, nrows):
    r_idx = lax.broadcasted_iota(jnp.int32, (nrows, LANES), 0)
    b_idx = lax.broadcasted_iota(jnp.int32, (nrows, LANES), 1)
    a_idx = lax.broadcasted_iota(jnp.int32, (nrows, nrows // LANES), 1)
    rr_idx = lax.broadcasted_iota(jnp.int32, (nrows, nrows // LANES), 0)
    sel = (rr_idx // LANES == a_idx).astype(jnp.float32)
    w = jax.lax.dot(sel, p, preferred_element_type=jnp.float32)
    w = w * (r_idx % LANES == b_idx).astype(jnp.float32)
    return jnp.sum(w, axis=1, keepdims=True)


def _partial_body(x_ref, pout_ref, xb_ref):
    xf = x_ref[...]
    s = jnp.sum(xf * xf, axis=1, keepdims=True)
    pout_ref[...] = _pack_rows(s, BLOCK_M)
    xb_ref[...] = xf.astype(jnp.bfloat16)


def _exchange_body(p_ref, out_ref, comm_ref, send_sems, recv_sems):
    me = lax.axis_index("i")

    barrier = pltpu.get_barrier_semaphore()
    for d in range(1, N_DEV):
        peer = (me + d) % N_DEV
        pl.semaphore_signal(
            barrier, inc=1,
            device_id=(peer,), device_id_type=pl.DeviceIdType.MESH,
        )
    pl.semaphore_wait(barrier, N_DEV - 1)

    sends = []
    for d in range(1, N_DEV):
        peer = (me + d) % N_DEV
        rdma = pltpu.make_async_remote_copy(
            src_ref=p_ref,
            dst_ref=comm_ref.at[me],
            send_sem=send_sems.at[d - 1],
            recv_sem=recv_sems.at[me],
            device_id=(peer,),
            device_id_type=pl.DeviceIdType.MESH,
        )
        rdma.start()
        sends.append(rdma)

    for d in range(1, N_DEV):
        src = (me - d + N_DEV) % N_DEV
        recv = pltpu.make_async_remote_copy(
            src_ref=p_ref,
            dst_ref=comm_ref.at[src],
            send_sem=send_sems.at[0],
            recv_sem=recv_sems.at[src],
            device_id=(me,),
            device_id_type=pl.DeviceIdType.MESH,
        )
        recv.wait_recv()

    total = p_ref[...]
    for peer in range(N_DEV):
        total = total + jnp.where(me == peer, 0.0, comm_ref[peer, :, :])

    for rdma in sends:
        rdma.wait_send()

    out_ref[...] = lax.rsqrt(total * (1.0 / (N_DEV * 2048.0)) + EPS)


def _scale_body(xb_ref, r_ref, g_ref, out_ref):
    u = _unpack_rows(r_ref[...], BLOCK_M).astype(jnp.bfloat16)
    out_ref[...] = xb_ref[...] * u * g_ref[...].astype(jnp.bfloat16)


def kernel(x, gamma):
    m, n_loc = x.shape
    nblk = m // BLOCK_M
    pk = BLOCK_M // LANES

    partial, xb = pl.pallas_call(
        _partial_body,
        grid=(nblk,),
        out_shape=(
            jax.ShapeDtypeStruct((m // LANES, LANES), jnp.float32),
            jax.ShapeDtypeStruct((m, n_loc), jnp.bfloat16),
        ),
        in_specs=[
            pl.BlockSpec((BLOCK_M, n_loc), lambda i: (i, 0),
                         memory_space=pltpu.VMEM),
        ],
        out_specs=(
            pl.BlockSpec((pk, LANES), lambda i: (i, 0),
                         memory_space=pltpu.VMEM),
            pl.BlockSpec((BLOCK_M, n_loc), lambda i: (i, 0),
                         memory_space=pltpu.VMEM),
        ),
    )(x)

    rrms = pl.pallas_call(
        _exchange_body,
        out_shape=jax.ShapeDtypeStruct((m // LANES, LANES), jnp.float32),
        in_specs=[pl.BlockSpec(memory_space=pltpu.VMEM)],
        out_specs=pl.BlockSpec(memory_space=pltpu.VMEM),
        scratch_shapes=[
            pltpu.VMEM((N_DEV, m // LANES, LANES), jnp.float32),
            pltpu.SemaphoreType.DMA((N_DEV - 1,)),
            pltpu.SemaphoreType.DMA((N_DEV,)),
        ],
        compiler_params=pltpu.CompilerParams(collective_id=0),
    )(partial)

    g2 = gamma.reshape(1, n_loc)

    out = pl.pallas_call(
        _scale_body,
        grid=(nblk,),
        out_shape=jax.ShapeDtypeStruct((m, n_loc), jnp.bfloat16),
        in_specs=[
            pl.BlockSpec((BLOCK_M, n_loc), lambda i: (i, 0),
                         memory_space=pltpu.VMEM),
            pl.BlockSpec((pk, LANES), lambda i: (i, 0),
                         memory_space=pltpu.VMEM),
            pl.BlockSpec((1, n_loc), lambda i: (0, 0),
                         memory_space=pltpu.VMEM),
        ],
        out_specs=pl.BlockSpec((BLOCK_M, n_loc), lambda i: (i, 0),
                               memory_space=pltpu.VMEM),
    )(xb, rrms, g2)
    return out


# device time: 60347 ns/iter; 1.0936x vs baseline; 1.0150x over previous
import jax
import jax.numpy as jnp
from jax import lax
from jax.experimental import pallas as pl
from jax.experimental.pallas import tpu as pltpu

N_DEV = 4
EPS = 1e-5
BLOCK_M = 1024
LANES = 128


def _pack_rows(s, nrows):
    r_idx = lax.broadcasted_iota(jnp.int32, (nrows, LANES), 0)
    b_idx = lax.broadcasted_iota(jnp.int32, (nrows, LANES), 1)
    masked = s * (r_idx % LANES == b_idx).astype(jnp.float32)
    a_idx = lax.broadcasted_iota(jnp.int32, (nrows // LANES, nrows), 0)
    rr_idx = lax.broadcasted_iota(jnp.int32, (nrows // LANES, nrows), 1)
    sel = (rr_idx // LANES == a_idx).astype(jnp.float32)
    return jax.lax.dot(sel, masked, preferred_element_type=jnp.float32)


def _unpack_rows(p, nrows):
    r_idx = lax.broadcasted_iota(jnp.int32, (nrows, LANES), 0)
    b_idx = lax.broadcasted_iota(jnp.int32, (nrows, LANES), 1)
    a_idx = lax.broadcasted_iota(jnp.int32, (nrows, nrows // LANES), 1)
    rr_idx = lax.broadcasted_iota(jnp.int32, (nrows, nrows // LANES), 0)
    sel = (rr_idx // LANES == a_idx).astype(jnp.float32)
    w = jax.lax.dot(sel, p, preferred_element_type=jnp.float32)
    w = w * (r_idx % LANES == b_idx).astype(jnp.float32)
    return jnp.sum(w, axis=1, keepdims=True)


def _partial_exchange_body(x_ref, rout_ref, xb_ref,
                           acc_ref, comm_ref, send_sems, recv_sems):
    i = pl.program_id(0)
    nblk = pl.num_programs(0)
    me = lax.axis_index("i")
    pk = BLOCK_M // LANES

    xf = x_ref[...]
    s = jnp.sum(xf * xf, axis=1, keepdims=True)
    acc_ref[pl.ds(i * pk, pk), :] = _pack_rows(s, BLOCK_M)
    xb_ref[...] = xf.astype(jnp.bfloat16)

    @pl.when(i == nblk - 1)
    def _():
        barrier = pltpu.get_barrier_semaphore()
        for d in range(1, N_DEV):
            peer = (me + d) % N_DEV
            pl.semaphore_signal(
                barrier, inc=1,
                device_id=(peer,), device_id_type=pl.DeviceIdType.MESH,
            )
        pl.semaphore_wait(barrier, N_DEV - 1)

        sends = []
        for d in range(1, N_DEV):
            peer = (me + d) % N_DEV
            rdma = pltpu.make_async_remote_copy(
                src_ref=acc_ref,
                dst_ref=comm_ref.at[me],
                send_sem=send_sems.at[d - 1],
                recv_sem=recv_sems.at[me],
                device_id=(peer,),
                device_id_type=pl.DeviceIdType.MESH,
            )
            rdma.start()
            sends.append(rdma)

        for d in range(1, N_DEV):
            src = (me - d + N_DEV) % N_DEV
            recv = pltpu.make_async_remote_copy(
                src_ref=acc_ref,
                dst_ref=comm_ref.at[src],
                send_sem=send_sems.at[0],
                recv_sem=recv_sems.at[src],
                device_id=(me,),
                device_id_type=pl.DeviceIdType.MESH,
            )
            recv.wait_recv()

        total = acc_ref[...]
        for peer in range(N_DEV):
            total = total + jnp.where(me == peer, 0.0, comm_ref[peer, :, :])

        for rdma in sends:
            rdma.wait_send()

        rout_ref[...] = lax.rsqrt(total * (1.0 / (N_DEV * 2048.0)) + EPS)


def _scale_body(xb_ref, r_ref, g_ref, out_ref):
    u = _unpack_rows(r_ref[...], BLOCK_M).astype(jnp.bfloat16)
    out_ref[...] = xb_ref[...] * u * g_ref[...].astype(jnp.bfloat16)


def kernel(x, gamma):
    m, n_loc = x.shape
    nblk = m // BLOCK_M
    pk = BLOCK_M // LANES

    rrms, xb = pl.pallas_call(
        _partial_exchange_body,
        grid=(nblk,),
        out_shape=(
            jax.ShapeDtypeStruct((m // LANES, LANES), jnp.float32),
            jax.ShapeDtypeStruct((m, n_loc), jnp.bfloat16),
        ),
        in_specs=[
            pl.BlockSpec((BLOCK_M, n_loc), lambda i: (i, 0),
                         memory_space=pltpu.VMEM),
        ],
        out_specs=(
            pl.BlockSpec((m // LANES, LANES), lambda i: (0, 0),
                         memory_space=pltpu.VMEM),
            pl.BlockSpec((BLOCK_M, n_loc), lambda i: (i, 0),
                         memory_space=pltpu.VMEM),
        ),
        scratch_shapes=[
            pltpu.VMEM((m // LANES, LANES), jnp.float32),
            pltpu.VMEM((N_DEV, m // LANES, LANES), jnp.float32),
            pltpu.SemaphoreType.DMA((N_DEV - 1,)),
            pltpu.SemaphoreType.DMA((N_DEV,)),
        ],
        compiler_params=pltpu.CompilerParams(collective_id=0),
    )(x)

    g2 = gamma.reshape(1, n_loc)

    out = pl.pallas_call(
        _scale_body,
        grid=(nblk,),
        out_shape=jax.ShapeDtypeStruct((m, n_loc), jnp.bfloat16),
        in_specs=[
            pl.BlockSpec((BLOCK_M, n_loc), lambda i: (i, 0),
                         memory_space=pltpu.VMEM),
            pl.BlockSpec((pk, LANES), lambda i: (i, 0),
                         memory_space=pltpu.VMEM),
            pl.BlockSpec((1, n_loc), lambda i: (0, 0),
                         memory_space=pltpu.VMEM),
        ],
        out_specs=pl.BlockSpec((BLOCK_M, n_loc), lambda i: (i, 0),
                               memory_space=pltpu.VMEM),
    )(xb, rrms, g2)
    return out
